# speedup vs baseline: 5.3096x; 5.3096x over previous
import numpy as np
import jax
import jax.numpy as jnp
from jax.experimental import pallas as pl
from jax.experimental.pallas import tpu as pltpu

N_FFT = 4096
N_FREQ = N_FFT // 2 + 1
TOP_K = 32
F_PAD = 2176
N_FB = 8
N_TB = 8
BISECT_ITERS = 30


def _make_tables(n, n_freq, f_pad):
    t = np.arange(n)
    f = np.arange(f_pad)
    ph = (np.outer(f, t) % n) * (2.0 * np.pi / n)
    cos = np.cos(ph)
    sin = np.sin(ph)
    cos[n_freq:] = 0.0
    sin[n_freq:] = 0.0
    w = np.full((f_pad, 1), 2.0 / n)
    w[0, 0] = 1.0 / n
    if n % 2 == 0 and n // 2 < f_pad:
        w[n // 2, 0] = 1.0 / n
    w[n_freq:] = 0.0
    icos = np.ascontiguousarray((cos * w).T)
    isin = np.ascontiguousarray((sin * w).T)
    return (cos.astype(np.float32), sin.astype(np.float32),
            icos.astype(np.float32), isin.astype(np.float32))


def _fwd_kernel(cos_ref, sin_ref, x_ref, re_ref, s_ref):
    xb = x_ref[0]
    dn = (((1,), (0,)), ((), ()))
    re_ref[0] = jax.lax.dot_general(
        cos_ref[...], xb, dn, precision=jax.lax.Precision.HIGHEST,
        preferred_element_type=jnp.float32)
    s_ref[0] = jax.lax.dot_general(
        sin_ref[...], xb, dn, precision=jax.lax.Precision.HIGHEST,
        preferred_element_type=jnp.float32)


def _make_mask_kernel(top_k):
    def _mask_kernel(re_ref, s_ref, mre_ref, ms_ref):
        re = re_ref[0]
        s = s_ref[0]
        mag = re * re + s * s
        hi = jnp.max(mag, axis=0, keepdims=True)
        lo = jnp.full_like(hi, -1.0)

        def body(_, carry):
            lo, hi = carry
            mid = 0.5 * (lo + hi)
            cnt = jnp.sum((mag > mid).astype(jnp.float32), axis=0,
                          keepdims=True)
            big = cnt >= top_k
            return jnp.where(big, mid, lo), jnp.where(big, hi, mid)

        lo, hi = jax.lax.fori_loop(0, BISECT_ITERS, body, (lo, hi))
        keep = mag > lo
        mre_ref[0] = jnp.where(keep, re, 0.0).astype(jnp.bfloat16)
        ms_ref[0] = jnp.where(keep, s, 0.0).astype(jnp.bfloat16)
    return _mask_kernel


def _inv_kernel(icos_ref, isin_ref, mre_ref, ms_ref, x_ref, xf_ref, res_ref):
    dn = (((1,), (0,)), ((), ()))
    acc = jax.lax.dot_general(icos_ref[...], mre_ref[0], dn,
                              preferred_element_type=jnp.float32)
    acc = acc + jax.lax.dot_general(isin_ref[...], ms_ref[0], dn,
                                    preferred_element_type=jnp.float32)
    xf_ref[0] = acc
    res_ref[0] = x_ref[0] - acc


def _pipeline(xp, cos, sin, icos, isin, top_k, interpret=False):
    bp, n, cp = xp.shape
    f_pad = cos.shape[0]
    fb = f_pad // N_FB
    tb = n // N_TB

    re, s = pl.pallas_call(
        _fwd_kernel,
        grid=(N_FB, bp),
        in_specs=[
            pl.BlockSpec((fb, n), lambda i, j: (i, 0)),
            pl.BlockSpec((fb, n), lambda i, j: (i, 0)),
            pl.BlockSpec((1, n, cp), lambda i, j: (j, 0, 0)),
        ],
        out_specs=[
            pl.BlockSpec((1, fb, cp), lambda i, j: (j, i, 0)),
            pl.BlockSpec((1, fb, cp), lambda i, j: (j, i, 0)),
        ],
        out_shape=[jax.ShapeDtypeStruct((bp, f_pad, cp), jnp.float32)] * 2,
        interpret=interpret,
    )(cos, sin, xp)

    mre, ms = pl.pallas_call(
        _make_mask_kernel(top_k),
        grid=(bp,),
        in_specs=[pl.BlockSpec((1, f_pad, cp), lambda i: (i, 0, 0))] * 2,
        out_specs=[pl.BlockSpec((1, f_pad, cp), lambda i: (i, 0, 0))] * 2,
        out_shape=[jax.ShapeDtypeStruct((bp, f_pad, cp), jnp.bfloat16)] * 2,
        interpret=interpret,
    )(re, s)

    xf, res = pl.pallas_call(
        _inv_kernel,
        grid=(N_TB, bp),
        in_specs=[
            pl.BlockSpec((tb, f_pad), lambda i, j: (i, 0)),
            pl.BlockSpec((tb, f_pad), lambda i, j: (i, 0)),
            pl.BlockSpec((1, f_pad, cp), lambda i, j: (j, 0, 0)),
            pl.BlockSpec((1, f_pad, cp), lambda i, j: (j, 0, 0)),
            pl.BlockSpec((1, tb, cp), lambda i, j: (j, i, 0)),
        ],
        out_specs=[pl.BlockSpec((1, tb, cp), lambda i, j: (j, i, 0))] * 2,
        out_shape=[jax.ShapeDtypeStruct((bp, n, cp), jnp.float32)] * 2,
        interpret=interpret,
    )(icos.astype(jnp.bfloat16), isin.astype(jnp.bfloat16), mre, ms, xp)
    return xf, res


_COS, _SIN, _ICOS, _ISIN = _make_tables(N_FFT, N_FREQ, F_PAD)


def kernel(x):
    b, n, c = x.shape
    bp = b // 2
    xp = x.reshape(2, bp, n, c).transpose(1, 2, 0, 3).reshape(bp, n, 2 * c)
    xf, res = _pipeline(xp, jnp.asarray(_COS), jnp.asarray(_SIN),
                        jnp.asarray(_ICOS), jnp.asarray(_ISIN), TOP_K)

    def unpair(y):
        return (y.reshape(bp, n, 2, c).transpose(2, 0, 1, 3)
                .reshape(b, n, c))

    return unpair(xf), unpair(res)


# speedup vs baseline: 6.8126x; 1.2831x over previous
import numpy as np
import jax
import jax.numpy as jnp
from jax.experimental import pallas as pl
from jax.experimental.pallas import tpu as pltpu

N_FFT = 4096
N_FREQ = N_FFT // 2 + 1
TOP_K = 32
F_PAD = 2176
N_FB = 8
N_TB = 8
BISECT_ITERS = 30


def _make_tables(n, n_freq, f_pad):
    t = np.arange(n)
    f = np.arange(f_pad)
    ph = (np.outer(f, t) % n) * (2.0 * np.pi / n)
    cos = np.cos(ph)
    sin = np.sin(ph)
    cos[n_freq:] = 0.0
    sin[n_freq:] = 0.0
    w = np.full((f_pad, 1), 2.0 / n)
    w[0, 0] = 1.0 / n
    if n % 2 == 0 and n // 2 < f_pad:
        w[n // 2, 0] = 1.0 / n
    w[n_freq:] = 0.0
    icos = np.ascontiguousarray((cos * w).T)
    isin = np.ascontiguousarray((sin * w).T)
    return (cos.astype(np.float32), sin.astype(np.float32),
            icos.astype(np.float32), isin.astype(np.float32))


def _fwd_kernel(cos_ref, sin_ref, x_ref, re_ref, s_ref):
    xb = jnp.concatenate([x_ref[0, 0], x_ref[1, 0]], axis=1)
    dn = (((1,), (0,)), ((), ()))
    re_ref[0] = jax.lax.dot_general(
        cos_ref[...], xb, dn, precision=jax.lax.Precision.HIGHEST,
        preferred_element_type=jnp.float32)
    s_ref[0] = jax.lax.dot_general(
        sin_ref[...], xb, dn, precision=jax.lax.Precision.HIGHEST,
        preferred_element_type=jnp.float32)


def _make_mask_kernel(top_k):
    def _mask_kernel(re_ref, s_ref, mre_ref, ms_ref):
        re = re_ref[0]
        s = s_ref[0]
        mag = re * re + s * s
        hi = jnp.max(mag, axis=0, keepdims=True)
        lo = jnp.full_like(hi, -1.0)

        def body(_, carry):
            lo, hi = carry
            mid = 0.5 * (lo + hi)
            cnt = jnp.sum((mag > mid).astype(jnp.float32), axis=0,
                          keepdims=True)
            big = cnt >= top_k
            return jnp.where(big, mid, lo), jnp.where(big, hi, mid)

        lo, hi = jax.lax.fori_loop(0, BISECT_ITERS, body, (lo, hi))
        keep = mag > lo
        mre_ref[0] = jnp.where(keep, re, 0.0).astype(jnp.bfloat16)
        ms_ref[0] = jnp.where(keep, s, 0.0).astype(jnp.bfloat16)
    return _mask_kernel


def _inv_kernel(icos_ref, isin_ref, mre_ref, ms_ref, x_ref, xf_ref, res_ref):
    dn = (((1,), (0,)), ((), ()))
    acc = jax.lax.dot_general(icos_ref[...], mre_ref[0], dn,
                              preferred_element_type=jnp.float32)
    acc = acc + jax.lax.dot_general(isin_ref[...], ms_ref[0], dn,
                                    preferred_element_type=jnp.float32)
    c = x_ref.shape[-1]
    xf_ref[0, 0] = acc[:, :c]
    xf_ref[1, 0] = acc[:, c:]
    res_ref[0, 0] = x_ref[0, 0] - acc[:, :c]
    res_ref[1, 0] = x_ref[1, 0] - acc[:, c:]


def _pipeline(x4d, cos, sin, icos, isin, top_k, interpret=False):
    _, bp, n, c = x4d.shape
    cp = 2 * c
    f_pad = cos.shape[0]
    fb = f_pad // N_FB
    tb = n // N_TB

    re, s = pl.pallas_call(
        _fwd_kernel,
        grid=(N_FB, bp),
        in_specs=[
            pl.BlockSpec((fb, n), lambda i, j: (i, 0)),
            pl.BlockSpec((fb, n), lambda i, j: (i, 0)),
            pl.BlockSpec((2, 1, n, c), lambda i, j: (0, j, 0, 0)),
        ],
        out_specs=[
            pl.BlockSpec((1, fb, cp), lambda i, j: (j, i, 0)),
            pl.BlockSpec((1, fb, cp), lambda i, j: (j, i, 0)),
        ],
        out_shape=[jax.ShapeDtypeStruct((bp, f_pad, cp), jnp.float32)] * 2,
        interpret=interpret,
    )(cos, sin, x4d)

    mre, ms = pl.pallas_call(
        _make_mask_kernel(top_k),
        grid=(bp,),
        in_specs=[pl.BlockSpec((1, f_pad, cp), lambda i: (i, 0, 0))] * 2,
        out_specs=[pl.BlockSpec((1, f_pad, cp), lambda i: (i, 0, 0))] * 2,
        out_shape=[jax.ShapeDtypeStruct((bp, f_pad, cp), jnp.bfloat16)] * 2,
        interpret=interpret,
    )(re, s)

    xf, res = pl.pallas_call(
        _inv_kernel,
        grid=(N_TB, bp),
        in_specs=[
            pl.BlockSpec((tb, f_pad), lambda i, j: (i, 0)),
            pl.BlockSpec((tb, f_pad), lambda i, j: (i, 0)),
            pl.BlockSpec((1, f_pad, cp), lambda i, j: (j, 0, 0)),
            pl.BlockSpec((1, f_pad, cp), lambda i, j: (j, 0, 0)),
            pl.BlockSpec((2, 1, tb, c), lambda i, j: (0, j, i, 0)),
        ],
        out_specs=[pl.BlockSpec((2, 1, tb, c), lambda i, j: (0, j, i, 0))] * 2,
        out_shape=[jax.ShapeDtypeStruct((2, bp, n, c), jnp.float32)] * 2,
        interpret=interpret,
    )(icos.astype(jnp.bfloat16), isin.astype(jnp.bfloat16), mre, ms, x4d)
    return xf, res


_COS, _SIN, _ICOS, _ISIN = _make_tables(N_FFT, N_FREQ, F_PAD)


def kernel(x):
    b, n, c = x.shape
    bp = b // 2
    x4d = x.reshape(2, bp, n, c)
    xf, res = _pipeline(x4d, jnp.asarray(_COS), jnp.asarray(_SIN),
                        jnp.asarray(_ICOS), jnp.asarray(_ISIN), TOP_K)
    return xf.reshape(b, n, c), res.reshape(b, n, c)


# speedup vs baseline: 7.3636x; 1.0809x over previous
import numpy as np
import jax
import jax.numpy as jnp
from jax.experimental import pallas as pl
from jax.experimental.pallas import tpu as pltpu

N_FFT = 4096
TOP_K = 32
F_HALF_PAD = 1088
N_FB = 8
N_TB = 8
BISECT_ITERS = 30


def _make_tables(n, f_half_pad):
    n2 = n // 2
    nf2 = n2 // 2 + 1
    u = np.arange(n2)
    f = np.arange(f_half_pad)
    ph = (np.outer(f, u) % n2) * (2.0 * np.pi / n2)
    cos_h = np.cos(ph)
    sin_h = np.sin(ph)
    cos_h[nf2:] = 0.0
    sin_h[nf2:] = 0.0

    tw_ph = f * (2.0 * np.pi / n)
    twc = np.repeat(np.cos(tw_ph)[:, None], 256, axis=1)
    tws = np.repeat(np.sin(tw_ph)[:, None], 256, axis=1)

    f_pad = 2 * f_half_pad
    bins = np.full(f_pad, -1)
    for r in range(nf2):
        bins[r] = r
    for g in range(n2 // 2):
        bins[f_half_pad + g] = n2 - g

    t = np.arange(n)
    icos = np.zeros((n, f_pad))
    isin = np.zeros((n, f_pad))
    for r in range(f_pad):
        fb = bins[r]
        if fb < 0:
            continue
        w = (1.0 / n) if (fb == 0 or fb == n // 2) else (2.0 / n)
        phr = (fb * t % n) * (2.0 * np.pi / n)
        icos[:, r] = w * np.cos(phr)
        isin[:, r] = w * np.sin(phr)

    return (cos_h.astype(np.float32), sin_h.astype(np.float32),
            twc.astype(np.float32), tws.astype(np.float32),
            icos.astype(np.float32), isin.astype(np.float32))


def _fwd_kernel(cos_ref, sin_ref, xe_ref, xo_ref,
                ree_ref, se_ref, reo_ref, so_ref):
    xe = jnp.concatenate([xe_ref[0, 0], xe_ref[1, 0]], axis=1)
    xo = jnp.concatenate([xo_ref[0, 0], xo_ref[1, 0]], axis=1)
    dn = (((1,), (0,)), ((), ()))
    hp = jax.lax.Precision.HIGHEST
    cosb = cos_ref[...]
    sinb = sin_ref[...]
    ree_ref[0] = jax.lax.dot_general(cosb, xe, dn, precision=hp,
                                     preferred_element_type=jnp.float32)
    se_ref[0] = jax.lax.dot_general(sinb, xe, dn, precision=hp,
                                    preferred_element_type=jnp.float32)
    reo_ref[0] = jax.lax.dot_general(cosb, xo, dn, precision=hp,
                                     preferred_element_type=jnp.float32)
    so_ref[0] = jax.lax.dot_general(sinb, xo, dn, precision=hp,
                                    preferred_element_type=jnp.float32)


def _make_mask_kernel(top_k, g_valid):
    def _mask_kernel(ree_ref, se_ref, reo_ref, so_ref, twc_ref, tws_ref,
                     mre_ref, ms_ref):
        ree = ree_ref[0]
        se = se_ref[0]
        reo = reo_ref[0]
        so = so_ref[0]
        c = twc_ref[...]
        s = tws_ref[...]
        t1 = c * reo - s * so
        t2 = c * so + s * reo
        re_l = ree + t1
        s_l = se + t2
        gmask = jax.lax.broadcasted_iota(
            jnp.int32, ree.shape, 0) < g_valid
        re_u = jnp.where(gmask, ree - t1, 0.0)
        s_u = jnp.where(gmask, t2 - se, 0.0)

        mag_l = re_l * re_l + s_l * s_l
        mag_u = re_u * re_u + s_u * s_u
        hi = jnp.maximum(jnp.max(mag_l, axis=0, keepdims=True),
                         jnp.max(mag_u, axis=0, keepdims=True))
        lo = jnp.full_like(hi, -1.0)

        def body(_, carry):
            lo, hi = carry
            mid = 0.5 * (lo + hi)
            cnt = (jnp.sum((mag_l > mid).astype(jnp.float32), axis=0,
                           keepdims=True)
                   + jnp.sum((mag_u > mid).astype(jnp.float32), axis=0,
                             keepdims=True))
            big = cnt >= top_k
            return jnp.where(big, mid, lo), jnp.where(big, hi, mid)

        lo, hi = jax.lax.fori_loop(0, BISECT_ITERS, body, (lo, hi))
        fh = ree.shape[0]
        mre_ref[0, :fh] = jnp.where(mag_l > lo, re_l, 0.0).astype(jnp.bfloat16)
        ms_ref[0, :fh] = jnp.where(mag_l > lo, s_l, 0.0).astype(jnp.bfloat16)
        mre_ref[0, fh:] = jnp.where(mag_u > lo, re_u, 0.0).astype(jnp.bfloat16)
        ms_ref[0, fh:] = jnp.where(mag_u > lo, s_u, 0.0).astype(jnp.bfloat16)
    return _mask_kernel


def _inv_kernel(icos_ref, isin_ref, mre_ref, ms_ref, x_ref, xf_ref, res_ref):
    dn = (((1,), (0,)), ((), ()))
    acc = jax.lax.dot_general(icos_ref[...], mre_ref[0], dn,
                              preferred_element_type=jnp.float32)
    acc = acc + jax.lax.dot_general(isin_ref[...], ms_ref[0], dn,
                                    preferred_element_type=jnp.float32)
    c = x_ref.shape[-1]
    xf_ref[0, 0] = acc[:, :c]
    xf_ref[1, 0] = acc[:, c:]
    res_ref[0, 0] = x_ref[0, 0] - acc[:, :c]
    res_ref[1, 0] = x_ref[1, 0] - acc[:, c:]


def _pipeline(x4d, cos_h, sin_h, twc, tws, icos, isin, top_k,
              n_fb=N_FB, interpret=False):
    _, bp, n, c = x4d.shape
    n2 = n // 2
    cp = 2 * c
    f_half = cos_h.shape[0]
    f_pad = 2 * f_half
    fb = f_half // n_fb
    tb = n // N_TB

    xe4 = x4d[:, :, 0::2, :]
    xo4 = x4d[:, :, 1::2, :]

    ree, se, reo, so = pl.pallas_call(
        _fwd_kernel,
        grid=(n_fb, bp),
        in_specs=[
            pl.BlockSpec((fb, n2), lambda i, j: (i, 0)),
            pl.BlockSpec((fb, n2), lambda i, j: (i, 0)),
            pl.BlockSpec((2, 1, n2, c), lambda i, j: (0, j, 0, 0)),
            pl.BlockSpec((2, 1, n2, c), lambda i, j: (0, j, 0, 0)),
        ],
        out_specs=[pl.BlockSpec((1, fb, cp), lambda i, j: (j, i, 0))] * 4,
        out_shape=[jax.ShapeDtypeStruct((bp, f_half, cp), jnp.float32)] * 4,
        interpret=interpret,
    )(cos_h, sin_h, xe4, xo4)

    mre, ms = pl.pallas_call(
        _make_mask_kernel(top_k, n2 // 2),
        grid=(bp,),
        in_specs=[pl.BlockSpec((1, f_half, cp), lambda i: (i, 0, 0))] * 4
        + [pl.BlockSpec((f_half, cp), lambda i: (0, 0))] * 2,
        out_specs=[pl.BlockSpec((1, f_pad, cp), lambda i: (i, 0, 0))] * 2,
        out_shape=[jax.ShapeDtypeStruct((bp, f_pad, cp), jnp.bfloat16)] * 2,
        interpret=interpret,
    )(ree, se, reo, so, twc, tws)

    xf, res = pl.pallas_call(
        _inv_kernel,
        grid=(N_TB, bp),
        in_specs=[
            pl.BlockSpec((tb, f_pad), lambda i, j: (i, 0)),
            pl.BlockSpec((tb, f_pad), lambda i, j: (i, 0)),
            pl.BlockSpec((1, f_pad, cp), lambda i, j: (j, 0, 0)),
            pl.BlockSpec((1, f_pad, cp), lambda i, j: (j, 0, 0)),
            pl.BlockSpec((2, 1, tb, c), lambda i, j: (0, j, i, 0)),
        ],
        out_specs=[pl.BlockSpec((2, 1, tb, c), lambda i, j: (0, j, i, 0))] * 2,
        out_shape=[jax.ShapeDtypeStruct((2, bp, n, c), jnp.float32)] * 2,
        interpret=interpret,
    )(icos.astype(jnp.bfloat16), isin.astype(jnp.bfloat16), mre, ms, x4d)
    return xf, res


_TABLES = _make_tables(N_FFT, F_HALF_PAD)


def kernel(x):
    b, n, c = x.shape
    bp = b // 2
    x4d = x.reshape(2, bp, n, c)
    xf, res = _pipeline(x4d, *(jnp.asarray(tbl) for tbl in _TABLES), TOP_K)
    return xf.reshape(b, n, c), res.reshape(b, n, c)


# speedup vs baseline: 10.9300x; 1.4843x over previous
import numpy as np
import jax
import jax.numpy as jnp
from jax.experimental import pallas as pl
from jax.experimental.pallas import tpu as pltpu

N_FFT = 4096
TOP_K = 32
F_HALF_PAD = 1088
N_FB = 8
N_TB = 8
BISECT_ITERS = 30


def _make_tables(n, f_half_pad):
    n2 = n // 2
    nf2 = n2 // 2 + 1
    u = np.arange(n2)
    f = np.arange(f_half_pad)
    ph = (np.outer(f, u) % n2) * (2.0 * np.pi / n2)
    cos_h = np.cos(ph)
    sin_h = np.sin(ph)
    cos_h[nf2:] = 0.0
    sin_h[nf2:] = 0.0

    tw_ph = f * (2.0 * np.pi / n)
    twc = np.repeat(np.cos(tw_ph)[:, None], 256, axis=1)
    tws = np.repeat(np.sin(tw_ph)[:, None], 256, axis=1)

    f_pad = 2 * f_half_pad
    bins = np.full(f_pad, -1)
    for r in range(nf2):
        bins[r] = r
    for g in range(n2 // 2):
        bins[f_half_pad + g] = n2 - g

    t = np.arange(n)
    icos = np.zeros((n, f_pad))
    isin = np.zeros((n, f_pad))
    for r in range(f_pad):
        fb = bins[r]
        if fb < 0:
            continue
        w = (1.0 / n) if (fb == 0 or fb == n // 2) else (2.0 / n)
        phr = (fb * t % n) * (2.0 * np.pi / n)
        icos[:, r] = w * np.cos(phr)
        isin[:, r] = w * np.sin(phr)

    return (cos_h.astype(np.float32), sin_h.astype(np.float32),
            twc.astype(np.float32), tws.astype(np.float32),
            icos.astype(np.float32), isin.astype(np.float32))


def _fwd_kernel(cos_ref, sin_ref, x_ref, re_ref, s_ref):
    xc = jnp.concatenate([x_ref[0, 0], x_ref[1, 0]], axis=1)
    n2 = xc.shape[0] // 2
    xeo = xc.reshape(n2, 2 * xc.shape[1])
    dn = (((1,), (0,)), ((), ()))
    hp = jax.lax.Precision.HIGHEST
    re_ref[0] = jax.lax.dot_general(cos_ref[...], xeo, dn, precision=hp,
                                    preferred_element_type=jnp.float32)
    s_ref[0] = jax.lax.dot_general(sin_ref[...], xeo, dn, precision=hp,
                                   preferred_element_type=jnp.float32)


def _make_mask_kernel(top_k, g_valid):
    def _mask_kernel(rp_ref, sp_ref, twc_ref, tws_ref, mre_ref, ms_ref):
        cp = rp_ref.shape[-1] // 2
        ree = rp_ref[0, :, :cp]
        reo = rp_ref[0, :, cp:]
        se = sp_ref[0, :, :cp]
        so = sp_ref[0, :, cp:]
        c = twc_ref[...]
        s = tws_ref[...]
        t1 = c * reo - s * so
        t2 = c * so + s * reo
        re_l = ree + t1
        s_l = se + t2
        gmask = jax.lax.broadcasted_iota(
            jnp.int32, ree.shape, 0) < g_valid
        re_u = jnp.where(gmask, ree - t1, 0.0)
        s_u = jnp.where(gmask, t2 - se, 0.0)

        mag_l = re_l * re_l + s_l * s_l
        mag_u = re_u * re_u + s_u * s_u
        hi = jnp.maximum(jnp.max(mag_l, axis=0, keepdims=True),
                         jnp.max(mag_u, axis=0, keepdims=True))
        lo = jnp.full_like(hi, -1.0)

        def body(_, carry):
            lo, hi = carry
            mid = 0.5 * (lo + hi)
            cnt = (jnp.sum((mag_l > mid).astype(jnp.float32), axis=0,
                           keepdims=True)
                   + jnp.sum((mag_u > mid).astype(jnp.float32), axis=0,
                             keepdims=True))
            big = cnt >= top_k
            return jnp.where(big, mid, lo), jnp.where(big, hi, mid)

        lo, hi = jax.lax.fori_loop(0, BISECT_ITERS, body, (lo, hi))
        fh = ree.shape[0]
        mre_ref[0, :fh] = jnp.where(mag_l > lo, re_l, 0.0).astype(jnp.bfloat16)
        ms_ref[0, :fh] = jnp.where(mag_l > lo, s_l, 0.0).astype(jnp.bfloat16)
        mre_ref[0, fh:] = jnp.where(mag_u > lo, re_u, 0.0).astype(jnp.bfloat16)
        ms_ref[0, fh:] = jnp.where(mag_u > lo, s_u, 0.0).astype(jnp.bfloat16)
    return _mask_kernel


def _inv_kernel(icos_ref, isin_ref, mre_ref, ms_ref, x_ref, xf_ref, res_ref):
    dn = (((1,), (0,)), ((), ()))
    acc = jax.lax.dot_general(icos_ref[...], mre_ref[0], dn,
                              preferred_element_type=jnp.float32)
    acc = acc + jax.lax.dot_general(isin_ref[...], ms_ref[0], dn,
                                    preferred_element_type=jnp.float32)
    c = x_ref.shape[-1]
    xf_ref[0, 0] = acc[:, :c]
    xf_ref[1, 0] = acc[:, c:]
    res_ref[0, 0] = x_ref[0, 0] - acc[:, :c]
    res_ref[1, 0] = x_ref[1, 0] - acc[:, c:]


def _pipeline(x4d, cos_h, sin_h, twc, tws, icos, isin, top_k,
              n_fb=N_FB, interpret=False):
    _, bp, n, c = x4d.shape
    n2 = n // 2
    cp = 2 * c
    f_half = cos_h.shape[0]
    f_pad = 2 * f_half
    fb = f_half // n_fb
    tb = n // N_TB

    rp, sp = pl.pallas_call(
        _fwd_kernel,
        grid=(bp,),
        in_specs=[
            pl.BlockSpec((f_half, n2), lambda j: (0, 0)),
            pl.BlockSpec((f_half, n2), lambda j: (0, 0)),
            pl.BlockSpec((2, 1, n, c), lambda j: (0, j, 0, 0)),
        ],
        out_specs=[pl.BlockSpec((1, f_half, 2 * cp), lambda j: (j, 0, 0))] * 2,
        out_shape=[jax.ShapeDtypeStruct((bp, f_half, 2 * cp),
                                        jnp.float32)] * 2,
        interpret=interpret,
    )(cos_h, sin_h, x4d)

    mre, ms = pl.pallas_call(
        _make_mask_kernel(top_k, n2 // 2),
        grid=(bp,),
        in_specs=[pl.BlockSpec((1, f_half, 2 * cp), lambda i: (i, 0, 0))] * 2
        + [pl.BlockSpec((f_half, cp), lambda i: (0, 0))] * 2,
        out_specs=[pl.BlockSpec((1, f_pad, cp), lambda i: (i, 0, 0))] * 2,
        out_shape=[jax.ShapeDtypeStruct((bp, f_pad, cp), jnp.bfloat16)] * 2,
        interpret=interpret,
    )(rp, sp, twc, tws)

    xf, res = pl.pallas_call(
        _inv_kernel,
        grid=(N_TB, bp),
        in_specs=[
            pl.BlockSpec((tb, f_pad), lambda i, j: (i, 0)),
            pl.BlockSpec((tb, f_pad), lambda i, j: (i, 0)),
            pl.BlockSpec((1, f_pad, cp), lambda i, j: (j, 0, 0)),
            pl.BlockSpec((1, f_pad, cp), lambda i, j: (j, 0, 0)),
            pl.BlockSpec((2, 1, tb, c), lambda i, j: (0, j, i, 0)),
        ],
        out_specs=[pl.BlockSpec((2, 1, tb, c), lambda i, j: (0, j, i, 0))] * 2,
        out_shape=[jax.ShapeDtypeStruct((2, bp, n, c), jnp.float32)] * 2,
        interpret=interpret,
    )(icos.astype(jnp.bfloat16), isin.astype(jnp.bfloat16), mre, ms, x4d)
    return xf, res


_TABLES = _make_tables(N_FFT, F_HALF_PAD)


def kernel(x):
    b, n, c = x.shape
    bp = b // 2
    x4d = x.reshape(2, bp, n, c)
    xf, res = _pipeline(x4d, *(jnp.asarray(tbl) for tbl in _TABLES), TOP_K)
    return xf.reshape(b, n, c), res.reshape(b, n, c)


# speedup vs baseline: 13.1141x; 1.1998x over previous
import numpy as np
import jax
import jax.numpy as jnp
from jax.experimental import pallas as pl
from jax.experimental.pallas import tpu as pltpu

N_FFT = 4096
TOP_K = 32
F_Q_PAD = 544
N_TB = 8
BISECT_ITERS = 30
_CONJ = (False, True, False, True)


def _section_bins(q, f_q_pad):
    h = q // 2
    fmap = np.full((4, f_q_pad), -1)
    for g in range(h + 1):
        fmap[0, g] = g
    for g in range(1, h):
        fmap[1, g] = q - g
    for g in range(h + 1):
        fmap[2, g] = q + g
    for g in range(h):
        fmap[3, g] = 2 * q - g
    return fmap


def _make_tables(n, f_q_pad, cp=256):
    q = n // 4
    nq = q // 2 + 1
    u = np.arange(q)
    m = np.arange(f_q_pad)
    ph = (np.outer(m, u) % q) * (2.0 * np.pi / q)
    cos_q = np.cos(ph)
    sin_q = np.sin(ph)
    cos_q[nq:] = 0.0
    sin_q[nq:] = 0.0

    fmap = _section_bins(q, f_q_pad)
    cw = np.zeros((4, 4, f_q_pad, cp))
    sw = np.zeros((4, 4, f_q_pad, cp))
    for s in range(4):
        valid = fmap[s] >= 0
        fr = np.where(valid, fmap[s], 0)
        for j in range(4):
            a = (fr * j % n) * (2.0 * np.pi / n)
            cw[s, j] = np.where(valid, np.cos(a), 0.0)[:, None]
            sw[s, j] = np.where(valid, np.sin(a), 0.0)[:, None]

    f_pad = 4 * f_q_pad
    t = np.arange(n)
    icos = np.zeros((n, f_pad))
    isin = np.zeros((n, f_pad))
    for s in range(4):
        for g in range(f_q_pad):
            fb = fmap[s, g]
            if fb < 0:
                continue
            r = s * f_q_pad + g
            w = (1.0 / n) if (fb == 0 or fb == n // 2) else (2.0 / n)
            phr = (fb * t % n) * (2.0 * np.pi / n)
            icos[:, r] = w * np.cos(phr)
            isin[:, r] = w * np.sin(phr)

    return (cos_q.astype(np.float32), sin_q.astype(np.float32),
            cw.astype(np.float32), sw.astype(np.float32),
            icos.astype(np.float32), isin.astype(np.float32))


def _fwd_kernel(cos_ref, sin_ref, x_ref, re_ref, s_ref):
    xc = jnp.concatenate([x_ref[0, 0], x_ref[1, 0]], axis=1)
    q = xc.shape[0] // 4
    xq = xc.reshape(q, 4 * xc.shape[1])
    dn = (((1,), (0,)), ((), ()))
    hp = jax.lax.Precision.HIGHEST
    re_ref[0] = jax.lax.dot_general(cos_ref[...], xq, dn, precision=hp,
                                    preferred_element_type=jnp.float32)
    s_ref[0] = jax.lax.dot_general(sin_ref[...], xq, dn, precision=hp,
                                   preferred_element_type=jnp.float32)


def _make_mask_kernel(top_k):
    def _mask_kernel(rp_ref, sp_ref, cw_ref, sw_ref, mre_ref, ms_ref):
        cp = rp_ref.shape[-1] // 4
        rea = [rp_ref[0, :, j * cp:(j + 1) * cp] for j in range(4)]
        sa = [sp_ref[0, :, j * cp:(j + 1) * cp] for j in range(4)]

        res = []
        for s in range(4):
            re_s = jnp.zeros_like(rea[0])
            s_s = jnp.zeros_like(rea[0])
            for j in range(4):
                c = cw_ref[s, j]
                w = sw_ref[s, j]
                if _CONJ[s]:
                    re_s = re_s + (c * rea[j] + w * sa[j])
                    s_s = s_s + (w * rea[j] - c * sa[j])
                else:
                    re_s = re_s + (c * rea[j] - w * sa[j])
                    s_s = s_s + (w * rea[j] + c * sa[j])
            res.append((re_s, s_s, re_s * re_s + s_s * s_s))

        hi = res[0][2].max(axis=0, keepdims=True)
        for s in range(1, 4):
            hi = jnp.maximum(hi, res[s][2].max(axis=0, keepdims=True))
        lo = jnp.full_like(hi, -1.0)

        def body(_, carry):
            lo, hi = carry
            mid = 0.5 * (lo + hi)
            cnt = sum(jnp.sum((mg > mid).astype(jnp.float32), axis=0,
                              keepdims=True) for _, _, mg in res)
            big = cnt >= top_k
            return jnp.where(big, mid, lo), jnp.where(big, hi, mid)

        lo, hi = jax.lax.fori_loop(0, BISECT_ITERS, body, (lo, hi))
        fq = rea[0].shape[0]
        for s, (re_s, s_s, mg) in enumerate(res):
            keep = mg > lo
            mre_ref[0, s * fq:(s + 1) * fq] = jnp.where(
                keep, re_s, 0.0).astype(jnp.bfloat16)
            ms_ref[0, s * fq:(s + 1) * fq] = jnp.where(
                keep, s_s, 0.0).astype(jnp.bfloat16)
    return _mask_kernel


def _inv_kernel(icos_ref, isin_ref, mre_ref, ms_ref, x_ref, xf_ref, res_ref):
    dn = (((1,), (0,)), ((), ()))
    acc = jax.lax.dot_general(icos_ref[...], mre_ref[0], dn,
                              preferred_element_type=jnp.float32)
    acc = acc + jax.lax.dot_general(isin_ref[...], ms_ref[0], dn,
                                    preferred_element_type=jnp.float32)
    c = x_ref.shape[-1]
    xf_ref[0, 0] = acc[:, :c]
    xf_ref[1, 0] = acc[:, c:]
    res_ref[0, 0] = x_ref[0, 0] - acc[:, :c]
    res_ref[1, 0] = x_ref[1, 0] - acc[:, c:]


def _pipeline(x4d, cos_q, sin_q, cw, sw, icos, isin, top_k, interpret=False):
    _, bp, n, c = x4d.shape
    q = n // 4
    cp = 2 * c
    f_q = cos_q.shape[0]
    f_pad = 4 * f_q
    tb = n // N_TB

    rp, sp = pl.pallas_call(
        _fwd_kernel,
        grid=(bp,),
        in_specs=[
            pl.BlockSpec((f_q, q), lambda j: (0, 0)),
            pl.BlockSpec((f_q, q), lambda j: (0, 0)),
            pl.BlockSpec((2, 1, n, c), lambda j: (0, j, 0, 0)),
        ],
        out_specs=[pl.BlockSpec((1, f_q, 4 * cp), lambda j: (j, 0, 0))] * 2,
        out_shape=[jax.ShapeDtypeStruct((bp, f_q, 4 * cp), jnp.float32)] * 2,
        interpret=interpret,
    )(cos_q, sin_q, x4d)

    mre, ms = pl.pallas_call(
        _make_mask_kernel(top_k),
        grid=(bp,),
        in_specs=[pl.BlockSpec((1, f_q, 4 * cp), lambda i: (i, 0, 0))] * 2
        + [pl.BlockSpec((4, 4, f_q, cp), lambda i: (0, 0, 0, 0))] * 2,
        out_specs=[pl.BlockSpec((1, f_pad, cp), lambda i: (i, 0, 0))] * 2,
        out_shape=[jax.ShapeDtypeStruct((bp, f_pad, cp), jnp.bfloat16)] * 2,
        interpret=interpret,
    )(rp, sp, cw, sw)

    xf, res = pl.pallas_call(
        _inv_kernel,
        grid=(N_TB, bp),
        in_specs=[
            pl.BlockSpec((tb, f_pad), lambda i, j: (i, 0)),
            pl.BlockSpec((tb, f_pad), lambda i, j: (i, 0)),
            pl.BlockSpec((1, f_pad, cp), lambda i, j: (j, 0, 0)),
            pl.BlockSpec((1, f_pad, cp), lambda i, j: (j, 0, 0)),
            pl.BlockSpec((2, 1, tb, c), lambda i, j: (0, j, i, 0)),
        ],
        out_specs=[pl.BlockSpec((2, 1, tb, c), lambda i, j: (0, j, i, 0))] * 2,
        out_shape=[jax.ShapeDtypeStruct((2, bp, n, c), jnp.float32)] * 2,
        interpret=interpret,
    )(icos.astype(jnp.bfloat16), isin.astype(jnp.bfloat16), mre, ms, x4d)
    return xf, res


_TABLES = _make_tables(N_FFT, F_Q_PAD)


def kernel(x):
    b, n, c = x.shape
    bp = b // 2
    x4d = x.reshape(2, bp, n, c)
    xf, res = _pipeline(x4d, *(jnp.asarray(tbl) for tbl in _TABLES), TOP_K)
    return xf.reshape(b, n, c), res.reshape(b, n, c)


# speedup vs baseline: 15.7940x; 1.2044x over previous
import numpy as np
import jax
import jax.numpy as jnp
from jax.experimental import pallas as pl
from jax.experimental.pallas import tpu as pltpu

N_FFT = 4096
TOP_K = 32
F_Q_PAD = 544
N_TB = 4
BISECT_ITERS = 30
_CONJ = (False, True, False, True)


def _section_bins(q, f_q_pad):
    h = q // 2
    fmap = np.full((4, f_q_pad), -1)
    for g in range(h + 1):
        fmap[0, g] = g
    for g in range(1, h):
        fmap[1, g] = q - g
    for g in range(h + 1):
        fmap[2, g] = q + g
    for g in range(h):
        fmap[3, g] = 2 * q - g
    return fmap


def _make_tables(n, f_q_pad, cp=256):
    q = n // 4
    nq = q // 2 + 1
    u = np.arange(q)
    m = np.arange(f_q_pad)
    ph = (np.outer(m, u) % q) * (2.0 * np.pi / q)
    cos_q = np.cos(ph)
    sin_q = np.sin(ph)
    cos_q[nq:] = 0.0
    sin_q[nq:] = 0.0

    fmap = _section_bins(q, f_q_pad)
    cw = np.zeros((4, 4, f_q_pad, cp))
    sw = np.zeros((4, 4, f_q_pad, cp))
    for s in range(4):
        valid = fmap[s] >= 0
        fr = np.where(valid, fmap[s], 0)
        for j in range(4):
            a = (fr * j % n) * (2.0 * np.pi / n)
            cw[s, j] = np.where(valid, np.cos(a), 0.0)[:, None]
            sw[s, j] = np.where(valid, np.sin(a), 0.0)[:, None]

    f_pad = 4 * f_q_pad
    t = np.arange(n)
    icos = np.zeros((n, f_pad))
    isin = np.zeros((n, f_pad))
    for s in range(4):
        for g in range(f_q_pad):
            fb = fmap[s, g]
            if fb < 0:
                continue
            r = s * f_q_pad + g
            w = (1.0 / n) if (fb == 0 or fb == n // 2) else (2.0 / n)
            phr = (fb * t % n) * (2.0 * np.pi / n)
            icos[:, r] = w * np.cos(phr)
            isin[:, r] = w * np.sin(phr)

    return (cos_q.astype(np.float32), sin_q.astype(np.float32),
            cw.astype(np.float32), sw.astype(np.float32),
            icos.astype(np.float32), isin.astype(np.float32))


def _make_fwdmask_kernel(top_k):
    def _mask_kernel(cos_ref, sin_ref, cw_ref, sw_ref, x_ref,
                     mre_ref, ms_ref):
        xc = jnp.concatenate([x_ref[0, 0], x_ref[1, 0]], axis=1)
        q = xc.shape[0] // 4
        xq = xc.reshape(q, 4 * xc.shape[1])
        dn = (((1,), (0,)), ((), ()))
        hp = jax.lax.Precision.HIGHEST
        rp = jax.lax.dot_general(cos_ref[...], xq, dn, precision=hp,
                                 preferred_element_type=jnp.float32)
        sp = jax.lax.dot_general(sin_ref[...], xq, dn, precision=hp,
                                 preferred_element_type=jnp.float32)
        cp = rp.shape[-1] // 4
        rea = [rp[:, j * cp:(j + 1) * cp] for j in range(4)]
        sa = [sp[:, j * cp:(j + 1) * cp] for j in range(4)]

        res = []
        for s in range(4):
            re_s = jnp.zeros_like(rea[0])
            s_s = jnp.zeros_like(rea[0])
            for j in range(4):
                c = cw_ref[s, j]
                w = sw_ref[s, j]
                if _CONJ[s]:
                    re_s = re_s + (c * rea[j] + w * sa[j])
                    s_s = s_s + (w * rea[j] - c * sa[j])
                else:
                    re_s = re_s + (c * rea[j] - w * sa[j])
                    s_s = s_s + (w * rea[j] + c * sa[j])
            res.append((re_s, s_s, re_s * re_s + s_s * s_s))

        hi = res[0][2].max(axis=0, keepdims=True)
        for s in range(1, 4):
            hi = jnp.maximum(hi, res[s][2].max(axis=0, keepdims=True))
        lo = jnp.full_like(hi, -1.0)

        def body(_, carry):
            lo, hi = carry
            mid = 0.5 * (lo + hi)
            cnt = sum(jnp.sum((mg > mid).astype(jnp.float32), axis=0,
                              keepdims=True) for _, _, mg in res)
            big = cnt >= top_k
            return jnp.where(big, mid, lo), jnp.where(big, hi, mid)

        lo, hi = jax.lax.fori_loop(0, BISECT_ITERS, body, (lo, hi))
        fq = rea[0].shape[0]
        for s, (re_s, s_s, mg) in enumerate(res):
            keep = mg > lo
            mre_ref[0, s * fq:(s + 1) * fq] = jnp.where(
                keep, re_s, 0.0).astype(jnp.bfloat16)
            ms_ref[0, s * fq:(s + 1) * fq] = jnp.where(
                keep, s_s, 0.0).astype(jnp.bfloat16)
    return _mask_kernel


def _inv_kernel(icos_ref, isin_ref, mre_ref, ms_ref, x_ref, xf_ref, res_ref):
    dn = (((1,), (0,)), ((), ()))
    acc = jax.lax.dot_general(icos_ref[...], mre_ref[0], dn,
                              preferred_element_type=jnp.float32)
    acc = acc + jax.lax.dot_general(isin_ref[...], ms_ref[0], dn,
                                    preferred_element_type=jnp.float32)
    c = x_ref.shape[-1]
    xf_ref[0, 0] = acc[:, :c]
    xf_ref[1, 0] = acc[:, c:]
    res_ref[0, 0] = x_ref[0, 0] - acc[:, :c]
    res_ref[1, 0] = x_ref[1, 0] - acc[:, c:]


def _pipeline(x4d, cos_q, sin_q, cw, sw, icos, isin, top_k, interpret=False):
    _, bp, n, c = x4d.shape
    q = n // 4
    cp = 2 * c
    f_q = cos_q.shape[0]
    f_pad = 4 * f_q
    tb = n // N_TB

    mre, ms = pl.pallas_call(
        _make_fwdmask_kernel(top_k),
        grid=(bp,),
        in_specs=[
            pl.BlockSpec((f_q, q), lambda j: (0, 0)),
            pl.BlockSpec((f_q, q), lambda j: (0, 0)),
            pl.BlockSpec((4, 4, f_q, cp), lambda j: (0, 0, 0, 0)),
            pl.BlockSpec((4, 4, f_q, cp), lambda j: (0, 0, 0, 0)),
            pl.BlockSpec((2, 1, n, c), lambda j: (0, j, 0, 0)),
        ],
        out_specs=[pl.BlockSpec((1, f_pad, cp), lambda j: (j, 0, 0))] * 2,
        out_shape=[jax.ShapeDtypeStruct((bp, f_pad, cp), jnp.bfloat16)] * 2,
        interpret=interpret,
    )(cos_q, sin_q, cw, sw, x4d)

    xf, res = pl.pallas_call(
        _inv_kernel,
        grid=(N_TB, bp),
        in_specs=[
            pl.BlockSpec((tb, f_pad), lambda i, j: (i, 0)),
            pl.BlockSpec((tb, f_pad), lambda i, j: (i, 0)),
            pl.BlockSpec((1, f_pad, cp), lambda i, j: (j, 0, 0)),
            pl.BlockSpec((1, f_pad, cp), lambda i, j: (j, 0, 0)),
            pl.BlockSpec((2, 1, tb, c), lambda i, j: (0, j, i, 0)),
        ],
        out_specs=[pl.BlockSpec((2, 1, tb, c), lambda i, j: (0, j, i, 0))] * 2,
        out_shape=[jax.ShapeDtypeStruct((2, bp, n, c), jnp.float32)] * 2,
        interpret=interpret,
    )(icos.astype(jnp.bfloat16), isin.astype(jnp.bfloat16), mre, ms, x4d)
    return xf, res


_TABLES = _make_tables(N_FFT, F_Q_PAD)


def kernel(x):
    b, n, c = x.shape
    bp = b // 2
    x4d = x.reshape(2, bp, n, c)
    xf, res = _pipeline(x4d, *(jnp.asarray(tbl) for tbl in _TABLES), TOP_K)
    return xf.reshape(b, n, c), res.reshape(b, n, c)
